# Initial kernel scaffold; baseline (speedup 1.0000x reference)
import jax
import jax.numpy as jnp
from jax.experimental import pallas as pl


def kernel(x):
    raise NotImplementedError("write your pallas kernel here")


# speedup vs baseline: 886.6116x; 886.6116x over previous
import functools

import jax
import jax.numpy as jnp
from jax.experimental import pallas as pl
from jax.experimental.pallas import tpu as pltpu

_CUM_P = 0.9
_B = 128
_N = 100000
_W = 8192
_NBLK = -(-_N // _W)
_BIG = 2**30


def _argmax_body(x_ref, maxv_ref, idx_ref, m_s, i_s):
    i = pl.program_id(0)
    blk = x_ref[...]
    col = jax.lax.broadcasted_iota(jnp.int32, blk.shape, 1) + i * _W
    blk = jnp.where(col < _N, blk, -1.0)
    bmax = jnp.max(blk, axis=1, keepdims=True)
    cand = jnp.where(blk == bmax, col, _BIG)
    bidx = jnp.min(cand, axis=1, keepdims=True)

    @pl.when(i == 0)
    def _():
        m_s[...] = bmax
        i_s[...] = bidx

    @pl.when(i > 0)
    def _():
        better = bmax > m_s[...]
        i_s[...] = jnp.where(better, bidx, i_s[...])
        m_s[...] = jnp.where(better, bmax, m_s[...])

    @pl.when(i == _NBLK - 1)
    def _():
        maxv_ref[...] = m_s[...]
        idx_ref[...] = i_s[...]


_argmax_call = pl.pallas_call(
    _argmax_body,
    grid=(_NBLK,),
    in_specs=[pl.BlockSpec((_B, _W), lambda i: (0, i))],
    out_specs=[
        pl.BlockSpec((_B, 1), lambda i: (0, 0)),
        pl.BlockSpec((_B, 1), lambda i: (0, 0)),
    ],
    out_shape=[
        jax.ShapeDtypeStruct((_B, 1), jnp.float32),
        jax.ShapeDtypeStruct((_B, 1), jnp.int32),
    ],
    scratch_shapes=[
        pltpu.VMEM((_B, 1), jnp.float32),
        pltpu.VMEM((_B, 1), jnp.int32),
    ],
)


def _full_nucleus_path(logits):
    order = jnp.argsort(-logits, axis=-1)
    sorted_logits = jnp.take_along_axis(logits, order, axis=-1)
    cumulative_probs = jnp.cumsum(sorted_logits, axis=-1)
    remove = cumulative_probs > _CUM_P
    remove = jnp.concatenate(
        [jnp.zeros_like(remove[..., :1]), remove[..., :-1]], axis=-1)
    rows = jnp.arange(logits.shape[0])[:, None]
    indices_to_remove = jnp.zeros_like(remove).at[rows, order].set(remove)
    probs = jnp.where(indices_to_remove, 0.0, logits)
    probs = probs * (1.0 / (probs.sum(axis=-1) + 1e-6))[..., None]
    return jax.random.categorical(jax.random.key(1), jnp.log(probs + 1e-20),
                                  axis=-1)


@jax.jit
def kernel(x):
    maxv, idx = _argmax_call(x)
    maxv = maxv[:, 0]
    idx = idx[:, 0]
    return jax.lax.cond(jnp.all(maxv > _CUM_P),
                        lambda: idx,
                        lambda: _full_nucleus_path(x))
